# Initial kernel scaffold; baseline (speedup 1.0000x reference)
import jax
import jax.numpy as jnp
from jax.experimental import pallas as pl


def kernel(coords, features, W, b):
    raise NotImplementedError("write your pallas kernel here")


# speedup vs baseline: 8.1305x; 8.1305x over previous
import functools

import jax
import jax.numpy as jnp
from jax.experimental import pallas as pl
from jax.experimental.pallas import tpu as pltpu
from jax.experimental.pallas import tpu_sc as plsc

K = 16
RA = 200
RC = 200
NC, NS = 2, 16
NW = NC * NS
LANES = 16


def _knn_body(cb_ref, ct_ref, idx_ref, dist_ref):
    rows = cb_ref.shape[0]
    npts = ct_ref.shape[1]
    cb = cb_ref[...]
    d2 = None
    for c in range(3):
        diff = cb[:, c:c + 1] - ct_ref[c:c + 1, :]
        sq = diff * diff
        d2 = sq if d2 is None else d2 + sq
    iota = jax.lax.broadcasted_iota(jnp.int32, (rows, npts), 1)
    big = jnp.float32(3.0e38)
    cand = d2
    vals = []
    idxs = []
    for _ in range(K):
        m = jnp.min(cand, axis=1, keepdims=True)
        sel = jnp.where(cand == m, iota, npts)
        j = jnp.min(sel, axis=1, keepdims=True)
        vals.append(m)
        idxs.append(j)
        cand = jnp.where(iota == j, big, cand)
    dist_ref[...] = jnp.sqrt(jnp.maximum(jnp.concatenate(vals, axis=1), 0.0))
    idx_ref[...] = jnp.concatenate(idxs, axis=1)


def _gather_body(xh, yh, zh, idxh, outx, outy, outz,
                 xv, yv, zv, idxv, gx, gy, gz):
    wid = jax.lax.axis_index("s") * NC + jax.lax.axis_index("c")
    pltpu.sync_copy(xh, xv)
    pltpu.sync_copy(yh, yv)
    pltpu.sync_copy(zh, zv)
    pltpu.sync_copy(idxh.at[wid], idxv)
    bpw = idxv.shape[0]

    def body(i, carry):
        sl = pl.ds(i * LANES, LANES)
        iv = idxv[sl]
        gx[sl] = plsc.load_gather(xv, [iv])
        gy[sl] = plsc.load_gather(yv, [iv])
        gz[sl] = plsc.load_gather(zv, [iv])
        return carry

    jax.lax.fori_loop(0, bpw // LANES, body, 0)
    pltpu.sync_copy(gx, outx.at[wid])
    pltpu.sync_copy(gy, outy.at[wid])
    pltpu.sync_copy(gz, outz.at[wid])


def _enc_body(cb_ref, f_ref, nx_ref, ny_ref, nz_ref, d_ref, misc_ref,
              out_ref):
    rows = cb_ref.shape[0]
    dh = f_ref.shape[1]
    cb = cb_ref[...]
    at = misc_ref[0:3, :]
    t1 = jnp.dot(cb, at, preferred_element_type=jnp.float32)
    t1 = t1 + misc_ref[7:8, :]
    term = t1[:, None, :]
    term = term + nx_ref[...][:, :, None] * misc_ref[3:4, :][None, :, :]
    term = term + ny_ref[...][:, :, None] * misc_ref[4:5, :][None, :, :]
    term = term + nz_ref[...][:, :, None] * misc_ref[5:6, :][None, :, :]
    term = term + d_ref[...][:, :, None] * misc_ref[6:7, :][None, :, :]
    out_ref[:, :, 0:dh] = term
    out_ref[:, :, dh:2 * dh] = jnp.broadcast_to(
        f_ref[...][:, None, :], (rows, K, dh))


def kernel(coords, features, W, b):
    n = coords.shape[0]
    dh = features.shape[1]

    knn_idx, knn_dist = pl.pallas_call(
        _knn_body,
        grid=(n // RA,),
        in_specs=[
            pl.BlockSpec((RA, 3), lambda i: (i, 0)),
            pl.BlockSpec((3, n), lambda i: (0, 0)),
        ],
        out_specs=[
            pl.BlockSpec((RA, K), lambda i: (i, 0)),
            pl.BlockSpec((RA, K), lambda i: (i, 0)),
        ],
        out_shape=[
            jax.ShapeDtypeStruct((n, K), jnp.int32),
            jax.ShapeDtypeStruct((n, K), jnp.float32),
        ],
    )(coords, coords.T)

    npad = ((n + 15) // 16) * 16
    total = n * K
    bpw = ((total + NW * 128 - 1) // (NW * 128)) * 128
    cpad = jnp.zeros((npad - n,), coords.dtype)
    xcol = jnp.concatenate([coords[:, 0], cpad])
    ycol = jnp.concatenate([coords[:, 1], cpad])
    zcol = jnp.concatenate([coords[:, 2], cpad])
    idx_flat = knn_idx.reshape(-1)
    idx_pad = jnp.concatenate(
        [idx_flat, jnp.zeros((NW * bpw - total,), jnp.int32)]
    ).reshape(NW, bpw)

    gfn = pl.kernel(
        _gather_body,
        out_type=[jax.ShapeDtypeStruct((NW, bpw), jnp.float32)] * 3,
        mesh=plsc.VectorSubcoreMesh(core_axis_name="c", subcore_axis_name="s"),
        compiler_params=pltpu.CompilerParams(needs_layout_passes=False),
        scratch_types=(
            [pltpu.VMEM((npad,), jnp.float32)] * 3
            + [pltpu.VMEM((bpw,), jnp.int32)]
            + [pltpu.VMEM((bpw,), jnp.float32)] * 3
        ),
    )
    nx, ny, nz = gfn(xcol, ycol, zcol, idx_pad)
    nbrx = nx.reshape(-1)[:total].reshape(n, K)
    nbry = ny.reshape(-1)[:total].reshape(n, K)
    nbrz = nz.reshape(-1)[:total].reshape(n, K)

    wc, wn, wd, wlast = W[:, 0:3], W[:, 3:6], W[:, 6:9], W[:, 9]
    misc = jnp.concatenate(
        [(wc + wd).T, (wn - wd).T, wlast.reshape(1, dh), b.reshape(1, dh)],
        axis=0)

    out = pl.pallas_call(
        _enc_body,
        grid=(n // RC,),
        in_specs=[
            pl.BlockSpec((RC, 3), lambda i: (i, 0)),
            pl.BlockSpec((RC, dh), lambda i: (i, 0)),
            pl.BlockSpec((RC, K), lambda i: (i, 0)),
            pl.BlockSpec((RC, K), lambda i: (i, 0)),
            pl.BlockSpec((RC, K), lambda i: (i, 0)),
            pl.BlockSpec((RC, K), lambda i: (i, 0)),
            pl.BlockSpec((8, dh), lambda i: (0, 0)),
        ],
        out_specs=pl.BlockSpec((RC, K, 2 * dh), lambda i: (i, 0, 0)),
        out_shape=jax.ShapeDtypeStruct((n, K, 2 * dh), jnp.float32),
    )(coords, features, nbrx, nbry, nbrz, knn_dist, misc)
    return out
